# baseline (speedup 1.0000x reference)
import jax
import jax.numpy as jnp
from jax.experimental import pallas as pl
from jax.experimental.pallas import tpu as pltpu

K = 512
D = 4
BETA = 0.25
BLK = 4096
CHUNK = 512


def _vq_kernel(x_ref, et_ref, e_ref, enc_ref, q_ref, loss_ref):
    i = pl.program_id(0)

    @pl.when(i == 0)
    def _init():
        loss_ref[0, 0] = jnp.float32(0.0)

    xc = x_ref[0]
    et = et_ref[...]
    e = e_ref[...]

    x2r = ((xc[0:1, :] * xc[0:1, :] + xc[1:2, :] * xc[1:2, :])
           + xc[2:3, :] * xc[2:3, :]) + xc[3:4, :] * xc[3:4, :]
    x2 = jnp.transpose(x2r)
    e2 = ((et[0:1, :] * et[0:1, :] + et[1:2, :] * et[1:2, :])
          + et[2:3, :] * et[2:3, :]) + et[3:4, :] * et[3:4, :]
    scores2 = jax.lax.dot_general(
        xc, et + et, (((0,), (0,)), ((), ())),
        preferred_element_type=jnp.float32)

    cols = jax.lax.broadcasted_iota(jnp.int32, (CHUNK, K), 1)
    loss_part = jnp.float32(0.0)
    for c in range(BLK // CHUNK):
        lo = c * CHUNK
        dist = (x2[lo:lo + CHUNK, :] + e2) - scores2[lo:lo + CHUNK, :]
        dmin = jnp.min(dist, axis=1, keepdims=True)
        idx = jnp.min(jnp.where(dist == dmin, cols, K),
                      axis=1).astype(jnp.int32)
        onehot = (cols == idx[:, None]).astype(jnp.float32)
        enc_ref[lo:lo + CHUNK, :] = onehot
        q_ref[lo:lo + CHUNK, :] = jnp.dot(
            onehot, e, preferred_element_type=jnp.float32)
        loss_part += jnp.sum(dmin)

    loss_ref[0, 0] += loss_part

    @pl.when(i == pl.num_programs(0) - 1)
    def _finish():
        n_el = pl.num_programs(0) * BLK * D
        loss_ref[0, 0] = loss_ref[0, 0] * ((1.0 + BETA) / n_el)


def kernel(x, emb_w):
    b, c, t, h, w = x.shape
    n = b * t * h * w
    xg = x.reshape(b, c, t * h * w)
    grid = n // BLK
    per_b = (t * h * w) // BLK

    enc, q, loss = pl.pallas_call(
        _vq_kernel,
        grid=(grid,),
        in_specs=[
            pl.BlockSpec((1, D, BLK), lambda i, pb=per_b: (i // pb, 0, i % pb)),
            pl.BlockSpec((D, K), lambda i: (0, 0)),
            pl.BlockSpec((K, D), lambda i: (0, 0)),
        ],
        out_specs=[
            pl.BlockSpec((BLK, K), lambda i: (i, 0)),
            pl.BlockSpec((BLK, D), lambda i: (i, 0)),
            pl.BlockSpec(memory_space=pltpu.SMEM),
        ],
        out_shape=[
            jax.ShapeDtypeStruct((n, K), jnp.float32),
            jax.ShapeDtypeStruct((n, D), jnp.float32),
            jax.ShapeDtypeStruct((1, 1), jnp.float32),
        ],
        compiler_params=pltpu.CompilerParams(
            vmem_limit_bytes=110 * 1024 * 1024),
    )(xg, emb_w.T, emb_w)

    q_out = jnp.transpose(q.reshape(b, t, h, w, c), (0, 4, 1, 2, 3))
    return (loss[0, 0], q_out, enc)


# speedup vs baseline: 1.0170x; 1.0170x over previous
import jax
import jax.numpy as jnp
from jax.experimental import pallas as pl
from jax.experimental.pallas import tpu as pltpu

K = 512
D = 4
BETA = 0.25
BLK = 4096


def _vq_kernel(x_ref, et_ref, e_ref, enc_ref, q_ref, loss_ref):
    i = pl.program_id(0)

    @pl.when(i == 0)
    def _init():
        loss_ref[0, 0] = jnp.float32(0.0)

    xc = x_ref[0]
    et = et_ref[...]
    e = e_ref[...]

    x2r = ((xc[0:1, :] * xc[0:1, :] + xc[1:2, :] * xc[1:2, :])
           + xc[2:3, :] * xc[2:3, :]) + xc[3:4, :] * xc[3:4, :]
    x2 = jnp.transpose(x2r)
    e2 = ((et[0:1, :] * et[0:1, :] + et[1:2, :] * et[1:2, :])
          + et[2:3, :] * et[2:3, :]) + et[3:4, :] * et[3:4, :]
    scores2 = jax.lax.dot_general(
        xc, et + et, (((0,), (0,)), ((), ())),
        preferred_element_type=jnp.float32)
    dist = (x2 + e2) - scores2

    cols = jax.lax.broadcasted_iota(jnp.int32, (BLK, K), 1)
    dmin = jnp.min(dist, axis=1, keepdims=True)
    idx = jnp.min(jnp.where(dist == dmin, cols, K), axis=1).astype(jnp.int32)
    onehot = (cols == idx[:, None]).astype(jnp.float32)
    enc_ref[...] = onehot

    qb = jnp.dot(onehot, e, preferred_element_type=jnp.float32)
    q_ref[...] = qb

    loss_ref[0, 0] += jnp.sum(dmin)

    @pl.when(i == pl.num_programs(0) - 1)
    def _finish():
        n_el = pl.num_programs(0) * BLK * D
        loss_ref[0, 0] = loss_ref[0, 0] * ((1.0 + BETA) / n_el)


def kernel(x, emb_w):
    b, c, t, h, w = x.shape
    n = b * t * h * w
    xg = x.reshape(b, c, t * h * w)
    grid = n // BLK
    per_b = (t * h * w) // BLK

    enc, q, loss = pl.pallas_call(
        _vq_kernel,
        grid=(grid,),
        in_specs=[
            pl.BlockSpec((1, D, BLK), lambda i, pb=per_b: (i // pb, 0, i % pb)),
            pl.BlockSpec((D, K), lambda i: (0, 0)),
            pl.BlockSpec((K, D), lambda i: (0, 0)),
        ],
        out_specs=[
            pl.BlockSpec((BLK, K), lambda i: (i, 0)),
            pl.BlockSpec((BLK, D), lambda i: (i, 0)),
            pl.BlockSpec(memory_space=pltpu.SMEM),
        ],
        out_shape=[
            jax.ShapeDtypeStruct((n, K), jnp.float32),
            jax.ShapeDtypeStruct((n, D), jnp.float32),
            jax.ShapeDtypeStruct((1, 1), jnp.float32),
        ],
        compiler_params=pltpu.CompilerParams(
            vmem_limit_bytes=110 * 1024 * 1024),
    )(xg, emb_w.T, emb_w)

    q_out = jnp.transpose(q.reshape(b, t, h, w, c), (0, 4, 1, 2, 3))
    return (loss[0, 0], q_out, enc)
